# baseline (speedup 1.0000x reference)
import functools

import jax
import jax.numpy as jnp
from jax import lax
from jax.experimental import pallas as pl
from jax.experimental.pallas import tpu as pltpu
from jax.experimental.pallas import tpu_sc as plsc

B, C, N, K = 8, 128, 4096, 16
L = 16
CB = 32
CP = CB // 2
NBLK = C // CB
NCH = 256
NT = N // NCH


def _body(x_hbm, idxt_hbm, out_hbm, tbl, raw, idxv, outb,
          sem_r, sem_i, sem_o):
    wid = lax.axis_index("s") * 2 + lax.axis_index("c")
    b = wid // NBLK
    c0 = (wid % NBLK) * CB

    idx_a0 = pltpu.async_copy(
        idxt_hbm.at[b, :, pl.ds(0, NCH)], idxv.at[0], sem_i.at[0])
    del idx_a0
    pltpu.async_copy(
        out_hbm.at[b, pl.ds(c0, CB), pl.ds(0, NCH)], outb.at[1],
        sem_o.at[1])

    def fire_pair(ci, slot):
        return (
            pltpu.async_copy(x_hbm.at[b, c0 + 2 * ci, :],
                             raw.at[2 * slot], sem_r.at[slot]),
            pltpu.async_copy(x_hbm.at[b, c0 + 2 * ci + 1, :],
                             raw.at[2 * slot + 1], sem_r.at[slot]),
        )

    pending = fire_pair(0, 0)
    for ci in range(CP):
        slot = ci & 1
        pending[0].wait()
        pending[1].wait()
        if ci + 1 < CP:
            pending = fire_pair(ci + 1, slot ^ 1)

        @plsc.parallel_loop(0, N // L, unroll=16)
        def pack_body(j2, ci=ci, slot=slot):
            a = raw[2 * slot, pl.ds(j2 * L, L)]
            bb = raw[2 * slot + 1, pl.ds(j2 * L, L)]
            w = plsc.bitcast(
                plsc.pack(a, bb, format=plsc.PackFormat.INTERLEAVED),
                jnp.int32)
            tbl[pl.ds(ci * N + j2 * L, L)] = w

    def compute_chunk(s, n0):
        @plsc.parallel_loop(0, NCH // L, unroll=2)
        def j_body(j):
            addrs = [idxv[s, k, pl.ds(j * L, L)] for k in range(K)]
            for ci in range(CP):
                m = [plsc.bitcast(plsc.load_gather(tbl, [addrs[k]]),
                                  jnp.bfloat16)
                     for k in range(K)]
                if ci + 1 < CP:
                    addrs = [a + N for a in addrs]
                while len(m) > 1:
                    m = [jnp.maximum(m[2 * a], m[2 * a + 1])
                         for a in range(len(m) // 2)]
                own = plsc.bitcast(tbl[pl.ds(ci * N + n0 + j * L, L)],
                                   jnp.bfloat16)
                best = jnp.maximum(m[0], own)
                ua, ub = plsc.unpack(best,
                                     format=plsc.PackFormat.INTERLEAVED)
                outb[s, 2 * ci, pl.ds(j * L, L)] = ua
                outb[s, 2 * ci + 1, pl.ds(j * L, L)] = ub

    def wait_idx(s):
        pltpu.make_async_copy(
            idxt_hbm.at[b, :, pl.ds(0, NCH)], idxv.at[s],
            sem_i.at[s]).wait()

    def wait_out(s):
        pltpu.make_async_copy(
            out_hbm.at[b, pl.ds(c0, CB), pl.ds(0, NCH)], outb.at[s],
            sem_o.at[s]).wait()

    def fire_idx(s, n0):
        pltpu.async_copy(idxt_hbm.at[b, :, pl.ds(n0, NCH)],
                         idxv.at[s], sem_i.at[s])

    def fire_out(s, n0):
        pltpu.async_copy(outb.at[s],
                         out_hbm.at[b, pl.ds(c0, CB), pl.ds(n0, NCH)],
                         sem_o.at[s])

    def loop_body(t2, carry):
        n0a = t2 * (2 * NCH)
        n0b = n0a + NCH
        wait_idx(0)
        fire_idx(1, n0b)
        compute_chunk(0, n0a)
        wait_out(1)
        fire_out(0, n0a)
        wait_idx(1)
        fire_idx(0, lax.rem(n0a + 2 * NCH, N))
        compute_chunk(1, n0b)
        wait_out(0)
        fire_out(1, n0b)
        return carry

    lax.fori_loop(0, NT // 2, loop_body, 0)
    wait_out(1)
    wait_idx(0)


_sc_call = functools.partial(
    pl.kernel,
    out_type=jax.ShapeDtypeStruct((B, C, N), jnp.float32),
    mesh=plsc.VectorSubcoreMesh(core_axis_name="c", subcore_axis_name="s"),
    compiler_params=pltpu.CompilerParams(needs_layout_passes=False,
                                        disable_bounds_checks=True),
    scratch_types=[
        pltpu.VMEM((CP * N,), jnp.int32),
        pltpu.VMEM((4, N), jnp.float32),
        pltpu.VMEM((2, K, NCH), jnp.int32),
        pltpu.VMEM((2, CB, NCH), jnp.float32),
        pltpu.SemaphoreType.DMA((2,)),
        pltpu.SemaphoreType.DMA((2,)),
        pltpu.SemaphoreType.DMA((2,)),
    ],
)(_body)


def kernel(x, batch_graph):
    idxt = jnp.transpose(batch_graph.astype(jnp.int32), (0, 2, 1))
    return _sc_call(x, idxt)


# speedup vs baseline: 1.0100x; 1.0100x over previous
import functools

import jax
import jax.numpy as jnp
from jax import lax
from jax.experimental import pallas as pl
from jax.experimental.pallas import tpu as pltpu
from jax.experimental.pallas import tpu_sc as plsc

B, C, N, K = 8, 128, 4096, 16
L = 16
CB = 32
CP = CB // 2
NBLK = C // CB
NCH = 256
NT = N // NCH


def _body(x_hbm, idxt_hbm, out_hbm, tbl, raw, idxv, outb,
          sem_r, sem_i, sem_o):
    wid = lax.axis_index("s") * 2 + lax.axis_index("c")
    b = wid // NBLK
    c0 = (wid % NBLK) * CB

    idx_a0 = pltpu.async_copy(
        idxt_hbm.at[b, :, pl.ds(0, NCH)], idxv.at[0], sem_i.at[0])
    del idx_a0
    pltpu.async_copy(
        out_hbm.at[b, pl.ds(c0, CB), pl.ds(0, NCH)], outb.at[1],
        sem_o.at[1])

    def fire_pair(ci, slot):
        return (
            pltpu.async_copy(x_hbm.at[b, c0 + 2 * ci, :],
                             raw.at[2 * slot], sem_r.at[slot]),
            pltpu.async_copy(x_hbm.at[b, c0 + 2 * ci + 1, :],
                             raw.at[2 * slot + 1], sem_r.at[slot]),
        )

    pending = fire_pair(0, 0)
    for ci in range(CP):
        slot = ci & 1
        pending[0].wait()
        pending[1].wait()
        if ci + 1 < CP:
            pending = fire_pair(ci + 1, slot ^ 1)

        @plsc.parallel_loop(0, N // L, unroll=8)
        def pack_body(j2, ci=ci, slot=slot):
            a = raw[2 * slot, pl.ds(j2 * L, L)]
            bb = raw[2 * slot + 1, pl.ds(j2 * L, L)]
            w = plsc.bitcast(
                plsc.pack(a, bb, format=plsc.PackFormat.INTERLEAVED),
                jnp.int32)
            tbl[pl.ds(ci * N + j2 * L, L)] = w

    def compute_chunk(s, n0):
        @plsc.parallel_loop(0, NCH // L, unroll=2)
        def j_body(j):
            addrs = [idxv[s, k, pl.ds(j * L, L)] for k in range(K)]
            for ci in range(CP):
                m = [plsc.bitcast(plsc.load_gather(tbl, [addrs[k]]),
                                  jnp.bfloat16)
                     for k in range(K)]
                if ci + 1 < CP:
                    addrs = [a + N for a in addrs]
                while len(m) > 1:
                    m = [jnp.maximum(m[2 * a], m[2 * a + 1])
                         for a in range(len(m) // 2)]
                own = plsc.bitcast(tbl[pl.ds(ci * N + n0 + j * L, L)],
                                   jnp.bfloat16)
                best = jnp.maximum(m[0], own)
                ua, ub = plsc.unpack(best,
                                     format=plsc.PackFormat.INTERLEAVED)
                outb[s, 2 * ci, pl.ds(j * L, L)] = ua
                outb[s, 2 * ci + 1, pl.ds(j * L, L)] = ub

    def wait_idx(s):
        pltpu.make_async_copy(
            idxt_hbm.at[b, :, pl.ds(0, NCH)], idxv.at[s],
            sem_i.at[s]).wait()

    def wait_out(s):
        pltpu.make_async_copy(
            out_hbm.at[b, pl.ds(c0, CB), pl.ds(0, NCH)], outb.at[s],
            sem_o.at[s]).wait()

    def fire_idx(s, n0):
        pltpu.async_copy(idxt_hbm.at[b, :, pl.ds(n0, NCH)],
                         idxv.at[s], sem_i.at[s])

    def fire_out(s, n0):
        pltpu.async_copy(outb.at[s],
                         out_hbm.at[b, pl.ds(c0, CB), pl.ds(n0, NCH)],
                         sem_o.at[s])

    def loop_body(t2, carry):
        n0a = t2 * (2 * NCH)
        n0b = n0a + NCH
        wait_idx(0)
        fire_idx(1, n0b)
        compute_chunk(0, n0a)
        wait_out(1)
        fire_out(0, n0a)
        wait_idx(1)
        fire_idx(0, lax.rem(n0a + 2 * NCH, N))
        compute_chunk(1, n0b)
        wait_out(0)
        fire_out(1, n0b)
        return carry

    lax.fori_loop(0, NT // 2, loop_body, 0)
    wait_out(1)
    wait_idx(0)


_sc_call = functools.partial(
    pl.kernel,
    out_type=jax.ShapeDtypeStruct((B, C, N), jnp.float32),
    mesh=plsc.VectorSubcoreMesh(core_axis_name="c", subcore_axis_name="s"),
    compiler_params=pltpu.CompilerParams(needs_layout_passes=False,
                                        disable_bounds_checks=True),
    scratch_types=[
        pltpu.VMEM((CP * N,), jnp.int32),
        pltpu.VMEM((4, N), jnp.float32),
        pltpu.VMEM((2, K, NCH), jnp.int32),
        pltpu.VMEM((2, CB, NCH), jnp.float32),
        pltpu.SemaphoreType.DMA((2,)),
        pltpu.SemaphoreType.DMA((2,)),
        pltpu.SemaphoreType.DMA((2,)),
    ],
)(_body)


def kernel(x, batch_graph):
    idxt = jnp.transpose(batch_graph.astype(jnp.int32), (0, 2, 1))
    return _sc_call(x, idxt)
